# baseline (speedup 1.0000x reference)
import functools

import jax
import jax.numpy as jnp
from jax import lax
from jax.experimental import pallas as pl
from jax.experimental.pallas import tpu as pltpu
from jax.experimental.pallas import tpu_sc as plsc

N = 10000
E = 320000
FEAT = 128
H = 128
B = 128
NUM_CONV = 3
EPS = 1e-5

NC = 2
NS = 16
NW = NC * NS
L = 16

EPT0 = E // NW
CHUNK = 128
DEG_PAD = 10016
TRASH = N

CUT = 3840
SA = 6400
ACC2 = 10368
TRASH_A = N - CUT
TRASH_B = SA + CUT
KCAP_A = 64
KCAP_B = 48
KTOT = KCAP_A + KCAP_B
LA = KCAP_A * CHUNK
LTOT = KTOT * CHUNK

_mesh = plsc.VectorSubcoreMesh(core_axis_name="c", subcore_axis_name="s")


@functools.partial(
    pl.kernel,
    out_type=(
        jax.ShapeDtypeStruct((NW, DEG_PAD), jnp.float32),
        jax.ShapeDtypeStruct((NW, KTOT, 2, CHUNK), jnp.int32),
        jax.ShapeDtypeStruct((NW, L), jnp.int32),
    ),
    mesh=_mesh,
    scratch_types=[
        pltpu.VMEM((DEG_PAD,), jnp.float32),
        pltpu.VMEM((EPT0,), jnp.int32),
        pltpu.VMEM((EPT0,), jnp.int32),
        pltpu.VMEM((LTOT,), jnp.int32),
        pltpu.VMEM((LTOT,), jnp.int32),
        pltpu.VMEM((KTOT, 2, CHUNK), jnp.int32),
        pltpu.VMEM((L,), jnp.int32),
    ],
    compiler_params=pltpu.CompilerParams(needs_layout_passes=False),
)
def _sc_prep(row_hbm, col_hbm, degp_hbm, rcl_hbm, cnt_hbm,
             deg_v, r_v, c_v, rowo, colo, rco, cnt_v):
    wid = lax.axis_index("s") * NC + lax.axis_index("c")
    base = wid * EPT0

    pltpu.sync_copy(row_hbm.at[pl.ds(base, EPT0)], r_v)
    pltpu.sync_copy(col_hbm.at[pl.ds(base, EPT0)], c_v)

    def zero(i, _):
        deg_v[pl.ds(i * L, L)] = jnp.zeros((L,), jnp.float32)
        return 0

    lax.fori_loop(0, DEG_PAD // L, zero, 0)

    def vec(i, offs):
        off_a, off_b = offs
        r = r_v[pl.ds(i * L, L)]
        c = c_v[pl.ds(i * L, L)]
        self_e = r == c
        w = jnp.where(self_e, 0.0, 1.0).astype(jnp.float32)
        plsc.addupdate_scatter(deg_v, [r], w)
        cadj = jnp.where(self_e, TRASH, c)
        m_b = cadj < CUT
        m_a = jnp.logical_not(m_b)
        plsc.store_compressed(rowo.at[pl.ds(off_a, L)], r, mask=m_a)
        plsc.store_compressed(colo.at[pl.ds(off_a, L)], cadj - CUT, mask=m_a)
        plsc.store_compressed(rowo.at[pl.ds(LA + off_b, L)], r, mask=m_b)
        plsc.store_compressed(colo.at[pl.ds(LA + off_b, L)], cadj + SA, mask=m_b)
        n_b = jnp.sum(m_b.astype(jnp.int32))
        return (off_a + (L - n_b), off_b + n_b)

    off_a, off_b = lax.fori_loop(0, EPT0 // L, vec, (0, 0))

    def pad_list(off, list_base, trash):
        k = (off + CHUNK - 1) // CHUNK
        k = jnp.maximum(k + (1 - k % 2), 3)
        zeros = jnp.zeros((L,), jnp.int32)
        trash_v = jnp.full((L,), trash, jnp.int32)

        def body(t, _):
            rowo[pl.ds(list_base + off + t * L, L)] = zeros
            colo[pl.ds(list_base + off + t * L, L)] = trash_v
            return 0

        lax.fori_loop(0, (k * CHUNK - off + L - 1) // L, body, 0)
        return k

    k_a = pad_list(off_a, 0, TRASH_A)
    k_b = pad_list(off_b, LA, TRASH_B)

    lane = lax.iota(jnp.int32, L)
    cnt_v[pl.ds(0, L)] = jnp.where(lane == 0, k_a,
                                   jnp.where(lane == 1, k_b, 0))

    def repack(j, _):
        for kk in range(CHUNK // L):
            rco[j, 0, pl.ds(kk * L, L)] = rowo[pl.ds(j * CHUNK + kk * L, L)]
            rco[j, 1, pl.ds(kk * L, L)] = colo[pl.ds(j * CHUNK + kk * L, L)]
        return 0

    lax.fori_loop(0, KTOT, repack, 0)

    pltpu.sync_copy(rco, rcl_hbm.at[wid])
    pltpu.sync_copy(cnt_v, cnt_hbm.at[wid])
    pltpu.sync_copy(deg_v, degp_hbm.at[wid])


@functools.partial(
    pl.kernel,
    out_type=jax.ShapeDtypeStruct((ACC2, H), jnp.float32),
    mesh=_mesh,
    scratch_types=[
        pltpu.VMEM_SHARED((ACC2, H), jnp.float32),
        pltpu.VMEM((2, CHUNK), jnp.int32),
        pltpu.VMEM((2, CHUNK), jnp.int32),
        pltpu.VMEM((2, L), jnp.int32),
        pltpu.VMEM((CHUNK, H), jnp.float32),
        pltpu.VMEM((CHUNK, H), jnp.float32),
        pltpu.SemaphoreType.DMA,
        pltpu.SemaphoreType.DMA,
        pltpu.SemaphoreType.DMA,
        pltpu.SemaphoreType.DMA,
    ],
    compiler_params=pltpu.CompilerParams(needs_layout_passes=False),
)
def _sc_prop(mp_hbm, rcl_hbm, cnt_hbm, out_hbm, acc,
             ib0, ib1, cbuf, g0, g1,
             gsem0, gsem1, isem0, isem1):
    cid = lax.axis_index("c")
    sid = lax.axis_index("s")
    chbase = jnp.where(cid == 0, 0, KCAP_A)
    sel = jnp.where(cid == 0, 0, 1)

    ibs = (ib0, ib1)
    gs = (g0, g1)
    gsems = (gsem0, gsem1)
    isems = (isem0, isem1)

    pltpu.sync_copy(cnt_hbm.at[pl.ds(2 * sid, 2)], cbuf)

    with jax.named_scope("acc_zero"):
        def zrow(i, _):
            g0[i // 8, pl.ds((i % 8) * L, L)] = jnp.zeros((L,), jnp.float32)
            return 0

        lax.fori_loop(0, CHUNK * (H // L), zrow, 0)

        zbase = sid * (ACC2 // NS)

        def zacc(i, _):
            pltpu.sync_copy(g0, acc.at[pl.ds(zbase + i * CHUNK, CHUNK)])
            return 0

        lax.fori_loop(0, ACC2 // NS // CHUNK, zacc, 0)
        pltpu.sync_copy(g0.at[pl.ds(0, ACC2 // NS % CHUNK)],
                        acc.at[pl.ds(zbase + ACC2 // NS - ACC2 // NS % CHUNK,
                                     ACC2 // NS % CHUNK)])
        plsc.subcore_barrier()

    def run_list(w, cnt):
        def prefetch_idx(j, slot):
            pltpu.async_copy(rcl_hbm.at[w, chbase + j], ibs[slot],
                             isems[slot])

        def gather(j, slot):
            pltpu.make_async_copy(rcl_hbm.at[w, chbase + j], ibs[slot],
                                  isems[slot]).wait()
            pltpu.async_copy(mp_hbm.at[ibs[slot].at[0]], gs[slot],
                             gsems[slot])

        def scatter(j, slot):
            pltpu.make_async_copy(
                mp_hbm.at[ibs[slot].at[0]], gs[slot], gsems[slot]).wait()
            pltpu.sync_copy(gs[slot], acc.at[ibs[slot].at[1]], add=True)

        prefetch_idx(0, 0)
        gather(0, 0)
        prefetch_idx(1, 1)

        def step(t, _):
            j = 2 * t + 1
            gather(j, 1)
            scatter(j - 1, 0)
            prefetch_idx(j + 1, 0)
            gather(j + 1, 0)
            scatter(j, 1)
            prefetch_idx(j + 2, 1)
            return 0

        lax.fori_loop(0, (cnt - 3) // 2, step, 0)
        gather(cnt - 2, 1)
        scatter(cnt - 3, 0)
        prefetch_idx(cnt - 1, 0)
        gather(cnt - 1, 0)
        scatter(cnt - 2, 1)
        scatter(cnt - 1, 0)

    with jax.named_scope("edges"):
        lane = lax.iota(jnp.int32, L)
        for widx in range(2):
            cvec = cbuf[widx, pl.ds(0, L)]
            cnt = jnp.sum(jnp.where(lane == sel, cvec, 0))
            run_list(2 * sid + widx, cnt)

    with jax.named_scope("dump"):
        plsc.subcore_barrier()

        @pl.when(cid == 0)
        def _():
            pltpu.sync_copy(acc.at[pl.ds(sid * (SA // NS), SA // NS)],
                            out_hbm.at[pl.ds(sid * (SA // NS), SA // NS)])

        @pl.when(cid != 0)
        def _():
            nb = (ACC2 - SA) // NS
            pltpu.sync_copy(acc.at[pl.ds(SA + sid * nb, nb)],
                            out_hbm.at[pl.ds(SA + sid * nb, nb)])


def _bn(v, g, b):
    mean = jnp.mean(v, axis=0, keepdims=True)
    var = jnp.mean((v - mean) ** 2, axis=0, keepdims=True)
    return (v - mean) * lax.rsqrt(var + EPS) * g + b


def _tc(body, out_shape):
    return pl.pallas_call(body, out_shape=out_shape)


def _k_dinv(degp_ref, out_ref):
    deg = jnp.sum(degp_ref[...], axis=0, keepdims=True) + 1.0
    out_ref[...] = lax.rsqrt(deg)


def _k_front(x_ref, g_ref, b_ref, w_ref, out_ref):
    h = _bn(x_ref[...], g_ref[...], b_ref[...])
    out_ref[...] = jnp.maximum(
        jnp.dot(h, w_ref[...], preferred_element_type=jnp.float32), 0.0)


def _k_pre(h_ref, g_ref, b_ref, w_ref, dinv_ref, out_ref):
    hn = _bn(h_ref[...], g_ref[...], b_ref[...])
    m = jnp.dot(hn, w_ref[...], preferred_element_type=jnp.float32)
    out_ref[...] = dinv_ref[...] * m


def _psum(p_ref):
    return jnp.concatenate(
        [p_ref[SA:SA + CUT, :], p_ref[0:N - CUT, :]], axis=0)


def _k_mid(h_ref, p_ref, mp_ref, dinv_ref, bias_ref, g_ref, b_ref, w_ref,
           hout_ref, mout_ref):
    dinv = dinv_ref[...]
    tot = dinv * (_psum(p_ref) + mp_ref[...]) + bias_ref[...]
    hnew = h_ref[...] + jnp.maximum(tot, 0.0)
    hout_ref[...] = hnew
    hn = _bn(hnew, g_ref[...], b_ref[...])
    m = jnp.dot(hn, w_ref[...], preferred_element_type=jnp.float32)
    mout_ref[...] = dinv * m


def _k_last(h_ref, p_ref, mp_ref, dinv_ref, bias_ref, hout_ref):
    dinv = dinv_ref[...]
    tot = dinv * (_psum(p_ref) + mp_ref[...]) + bias_ref[...]
    hout_ref[...] = h_ref[...] + jnp.maximum(tot, 0.0)


def _k_head(h_ref, batch_ref, fg_ref, fb_ref, wfc_ref, bfc_ref,
            hg_ref, hb_ref, wcl_ref, bcl_ref, out_ref):
    onehot = (batch_ref[...] == lax.broadcasted_iota(jnp.int32, (N, B), 1)
              ).astype(jnp.float32)
    xg = lax.dot_general(onehot, h_ref[...],
                         dimension_numbers=(((0,), (0,)), ((), ())),
                         preferred_element_type=jnp.float32)
    z = _bn(xg, fg_ref[...], fb_ref[...])
    z = jnp.maximum(
        jnp.dot(z, wfc_ref[...], preferred_element_type=jnp.float32)
        + bfc_ref[...], 0.0)
    z = _bn(z, hg_ref[...], hb_ref[...])
    u = jnp.dot(z, wcl_ref[...],
                preferred_element_type=jnp.float32) + bcl_ref[...]
    umax = jnp.max(u, axis=-1, keepdims=True)
    lse = jnp.log(jnp.sum(jnp.exp(u - umax), axis=-1, keepdims=True)) + umax
    out_ref[...] = u - lse


def kernel(x, edge_index, batch, bn_feat_g, bn_feat_b, W_feat, conv_bn_g,
           conv_bn_b, conv_W, conv_b, bn_fc_g, bn_fc_b, W_fc, b_fc,
           bn_hidden_g, bn_hidden_b, W_class, b_class):
    f32 = jnp.float32
    degp, rcl, cnts = _sc_prep(edge_index[0], edge_index[1])

    dinv_full = _tc(_k_dinv, jax.ShapeDtypeStruct((1, DEG_PAD), f32))(degp)
    dinv = dinv_full[0, :N].reshape(N, 1)

    r2 = lambda a: a.reshape(1, -1)
    h = _tc(_k_front, jax.ShapeDtypeStruct((N, H), f32))(
        x, r2(bn_feat_g), r2(bn_feat_b), W_feat)

    mp = _tc(_k_pre, jax.ShapeDtypeStruct((N, H), f32))(
        h, r2(conv_bn_g[0]), r2(conv_bn_b[0]), conv_W[0], dinv)

    for i in range(NUM_CONV):
        p = _sc_prop(mp, rcl, cnts)
        if i < NUM_CONV - 1:
            h, mp = _tc(
                _k_mid,
                (jax.ShapeDtypeStruct((N, H), f32),
                 jax.ShapeDtypeStruct((N, H), f32)),
            )(h, p, mp, dinv, r2(conv_b[i]), r2(conv_bn_g[i + 1]),
              r2(conv_bn_b[i + 1]), conv_W[i + 1])
        else:
            h = _tc(_k_last, jax.ShapeDtypeStruct((N, H), f32))(
                h, p, mp, dinv, r2(conv_b[i]))

    out = _tc(_k_head, jax.ShapeDtypeStruct((B, H), f32))(
        h, batch.reshape(N, 1), r2(bn_fc_g), r2(bn_fc_b), W_fc, r2(b_fc),
        r2(bn_hidden_g), r2(bn_hidden_b), W_class, r2(b_class))
    return out


# speedup vs baseline: 1.5990x; 1.5990x over previous
import functools

import jax
import jax.numpy as jnp
from jax import lax
from jax.experimental import pallas as pl
from jax.experimental.pallas import tpu as pltpu
from jax.experimental.pallas import tpu_sc as plsc

N = 10000
E = 320000
FEAT = 128
H = 128
B = 128
NUM_CONV = 3
EPS = 1e-5

NC = 2
NS = 16
NW = NC * NS
L = 16

E_PAD = 323584
EPT = E_PAD // NW
CHUNK = 128
NCH = EPT // CHUNK
NCHT = E_PAD // CHUNK
K_A = 139
K_B = NCHT // NS - K_A
ACC_ROWS = 10240
ROWS_PT = ACC_ROWS // NS
DEG_PAD = 10016
TRASH = N

_mesh = plsc.VectorSubcoreMesh(core_axis_name="c", subcore_axis_name="s")


@functools.partial(
    pl.kernel,
    out_type=(
        jax.ShapeDtypeStruct((NW, DEG_PAD), jnp.float32),
        jax.ShapeDtypeStruct((NCHT, 2, CHUNK), jnp.int32),
    ),
    mesh=_mesh,
    scratch_types=[
        pltpu.VMEM((DEG_PAD,), jnp.float32),
        pltpu.VMEM((NCH, 2, CHUNK), jnp.int32),
        pltpu.VMEM((NCH, 2, CHUNK), jnp.int32),
    ],
    compiler_params=pltpu.CompilerParams(needs_layout_passes=False),
)
def _sc_deg(rc_hbm, degp_hbm, rcadj_hbm, deg_v, rc_v, out_v):
    wid = lax.axis_index("s") * NC + lax.axis_index("c")

    pltpu.sync_copy(rc_hbm.at[pl.ds(wid * NCH, NCH)], rc_v)

    def zero(i, _):
        deg_v[pl.ds(i * L, L)] = jnp.zeros((L,), jnp.float32)
        return 0

    lax.fori_loop(0, DEG_PAD // L, zero, 0)

    def chunk(j, _):
        def vec(i, _):
            r = rc_v[j, 0, pl.ds(i * L, L)]
            c = rc_v[j, 1, pl.ds(i * L, L)]
            self_e = r == c
            w = jnp.where(self_e, 0.0, 1.0).astype(jnp.float32)
            plsc.addupdate_scatter(deg_v, [r], w)
            out_v[j, 0, pl.ds(i * L, L)] = r
            out_v[j, 1, pl.ds(i * L, L)] = jnp.where(self_e, TRASH, c)
            return 0

        lax.fori_loop(0, CHUNK // L, vec, 0)
        return 0

    lax.fori_loop(0, NCH, chunk, 0)
    pltpu.sync_copy(out_v, rcadj_hbm.at[pl.ds(wid * NCH, NCH)])
    pltpu.sync_copy(deg_v, degp_hbm.at[wid])


@functools.partial(
    pl.kernel,
    out_type=jax.ShapeDtypeStruct((NC, ACC_ROWS, H), jnp.float32),
    mesh=_mesh,
    scratch_types=[
        pltpu.VMEM_SHARED((ACC_ROWS, H), jnp.float32),
        pltpu.VMEM((2, CHUNK), jnp.int32),
        pltpu.VMEM((2, CHUNK), jnp.int32),
        pltpu.VMEM((CHUNK, H), jnp.float32),
        pltpu.VMEM((CHUNK, H), jnp.float32),
        pltpu.SemaphoreType.DMA,
        pltpu.SemaphoreType.DMA,
        pltpu.SemaphoreType.DMA,
        pltpu.SemaphoreType.DMA,
    ],
    compiler_params=pltpu.CompilerParams(needs_layout_passes=False),
)
def _sc_prop(mp_hbm, rc_hbm, out_hbm, acc, i0, i1, g0, g1,
             gsem0, gsem1, isem0, isem1):
    cid = lax.axis_index("c")
    sid = lax.axis_index("s")
    cnt = jnp.where(cid == 0, K_A, K_B)
    start = jnp.where(cid == 0, sid * K_A, NS * K_A + sid * K_B)

    ibs = (i0, i1)
    gs = (g0, g1)
    gsems = (gsem0, gsem1)
    isems = (isem0, isem1)

    with jax.named_scope("acc_zero"):
        def zrow(i, _):
            g0[i // 8, pl.ds((i % 8) * L, L)] = jnp.zeros((L,), jnp.float32)
            return 0

        lax.fori_loop(0, CHUNK * (H // L), zrow, 0)

        def zacc(i, _):
            pltpu.sync_copy(
                g0, acc.at[pl.ds(sid * ROWS_PT + i * CHUNK, CHUNK)])
            return 0

        lax.fori_loop(0, ROWS_PT // CHUNK, zacc, 0)
        plsc.subcore_barrier()

    def prefetch_idx(j, slot):
        pltpu.async_copy(rc_hbm.at[start + j], ibs[slot], isems[slot])

    def gather(j, slot):
        pltpu.make_async_copy(rc_hbm.at[start + j], ibs[slot],
                              isems[slot]).wait()
        pltpu.async_copy(mp_hbm.at[ibs[slot].at[0]], gs[slot], gsems[slot])

    def scatter(j, slot):
        pltpu.make_async_copy(
            mp_hbm.at[ibs[slot].at[0]], gs[slot], gsems[slot]).wait()
        pltpu.sync_copy(gs[slot], acc.at[ibs[slot].at[1]], add=True)

    with jax.named_scope("edges"):
        prefetch_idx(0, 0)
        gather(0, 0)
        prefetch_idx(1, 1)

        def step(t, _):
            j = 2 * t + 1
            gather(j, 1)
            scatter(j - 1, 0)
            prefetch_idx(j + 1, 0)
            gather(j + 1, 0)
            scatter(j, 1)
            prefetch_idx(j + 2, 1)
            return 0

        lax.fori_loop(0, (cnt - 3) // 2, step, 0)
        gather(cnt - 2, 1)
        scatter(cnt - 3, 0)
        prefetch_idx(cnt - 1, 0)
        gather(cnt - 1, 0)
        scatter(cnt - 2, 1)
        scatter(cnt - 1, 0)

    with jax.named_scope("dump"):
        plsc.subcore_barrier()

        def dmp(i, _):
            r = sid * ROWS_PT + i * CHUNK
            pltpu.sync_copy(acc.at[pl.ds(r, CHUNK)], g0)
            pltpu.sync_copy(g0, out_hbm.at[cid, pl.ds(r, CHUNK)])
            return 0

        lax.fori_loop(0, ROWS_PT // CHUNK, dmp, 0)


def _bn(v, g, b):
    mean = jnp.mean(v, axis=0, keepdims=True)
    var = jnp.mean((v - mean) ** 2, axis=0, keepdims=True)
    return (v - mean) * lax.rsqrt(var + EPS) * g + b


def _tc(body, out_shape):
    return pl.pallas_call(body, out_shape=out_shape)


def _k_dinv(degp_ref, out_ref):
    deg = jnp.sum(degp_ref[...], axis=0, keepdims=True) + 1.0
    out_ref[...] = lax.rsqrt(deg)


def _k_front(x_ref, g_ref, b_ref, w_ref, out_ref):
    h = _bn(x_ref[...], g_ref[...], b_ref[...])
    out_ref[...] = jnp.maximum(
        jnp.dot(h, w_ref[...], preferred_element_type=jnp.float32), 0.0)


def _k_pre(h_ref, g_ref, b_ref, w_ref, dinv_ref, out_ref):
    hn = _bn(h_ref[...], g_ref[...], b_ref[...])
    m = jnp.dot(hn, w_ref[...], preferred_element_type=jnp.float32)
    out_ref[...] = dinv_ref[...] * m


def _k_mid(h_ref, p_ref, mp_ref, dinv_ref, bias_ref, g_ref, b_ref, w_ref,
           hout_ref, mout_ref):
    dinv = dinv_ref[...]
    psum = p_ref[0, :N, :] + p_ref[1, :N, :]
    tot = dinv * (psum + mp_ref[...]) + bias_ref[...]
    hnew = h_ref[...] + jnp.maximum(tot, 0.0)
    hout_ref[...] = hnew
    hn = _bn(hnew, g_ref[...], b_ref[...])
    m = jnp.dot(hn, w_ref[...], preferred_element_type=jnp.float32)
    mout_ref[...] = dinv * m


def _k_last(h_ref, p_ref, mp_ref, dinv_ref, bias_ref, hout_ref):
    dinv = dinv_ref[...]
    psum = p_ref[0, :N, :] + p_ref[1, :N, :]
    tot = dinv * (psum + mp_ref[...]) + bias_ref[...]
    hout_ref[...] = h_ref[...] + jnp.maximum(tot, 0.0)


def _k_head(h_ref, batch_ref, fg_ref, fb_ref, wfc_ref, bfc_ref,
            hg_ref, hb_ref, wcl_ref, bcl_ref, out_ref):
    onehot = (batch_ref[...] == lax.broadcasted_iota(jnp.int32, (N, B), 1)
              ).astype(jnp.float32)
    xg = lax.dot_general(onehot, h_ref[...],
                         dimension_numbers=(((0,), (0,)), ((), ())),
                         preferred_element_type=jnp.float32)
    z = _bn(xg, fg_ref[...], fb_ref[...])
    z = jnp.maximum(
        jnp.dot(z, wfc_ref[...], preferred_element_type=jnp.float32)
        + bfc_ref[...], 0.0)
    z = _bn(z, hg_ref[...], hb_ref[...])
    u = jnp.dot(z, wcl_ref[...],
                preferred_element_type=jnp.float32) + bcl_ref[...]
    umax = jnp.max(u, axis=-1, keepdims=True)
    lse = jnp.log(jnp.sum(jnp.exp(u - umax), axis=-1, keepdims=True)) + umax
    out_ref[...] = u - lse


def kernel(x, edge_index, batch, bn_feat_g, bn_feat_b, W_feat, conv_bn_g,
           conv_bn_b, conv_W, conv_b, bn_fc_g, bn_fc_b, W_fc, b_fc,
           bn_hidden_g, bn_hidden_b, W_class, b_class):
    f32 = jnp.float32
    pad = jnp.zeros((E_PAD - E,), jnp.int32)
    rowp = jnp.concatenate([edge_index[0], pad]).reshape(NCHT, CHUNK)
    colp = jnp.concatenate([edge_index[1], pad]).reshape(NCHT, CHUNK)
    rc = jnp.stack([rowp, colp], axis=1)

    degp, rcadj = _sc_deg(rc)

    dinv_full = _tc(_k_dinv, jax.ShapeDtypeStruct((1, DEG_PAD), f32))(degp)
    dinv = dinv_full[0, :N].reshape(N, 1)

    r2 = lambda a: a.reshape(1, -1)
    h = _tc(_k_front, jax.ShapeDtypeStruct((N, H), f32))(
        x, r2(bn_feat_g), r2(bn_feat_b), W_feat)

    mp = _tc(_k_pre, jax.ShapeDtypeStruct((N, H), f32))(
        h, r2(conv_bn_g[0]), r2(conv_bn_b[0]), conv_W[0], dinv)

    for i in range(NUM_CONV):
        p = _sc_prop(mp, rcadj)
        if i < NUM_CONV - 1:
            h, mp = _tc(
                _k_mid,
                (jax.ShapeDtypeStruct((N, H), f32),
                 jax.ShapeDtypeStruct((N, H), f32)),
            )(h, p, mp, dinv, r2(conv_b[i]), r2(conv_bn_g[i + 1]),
              r2(conv_bn_b[i + 1]), conv_W[i + 1])
        else:
            h = _tc(_k_last, jax.ShapeDtypeStruct((N, H), f32))(
                h, p, mp, dinv, r2(conv_b[i]))

    out = _tc(_k_head, jax.ShapeDtypeStruct((B, H), f32))(
        h, batch.reshape(N, 1), r2(bn_fc_g), r2(bn_fc_b), W_fc, r2(b_fc),
        r2(bn_hidden_g), r2(bn_hidden_b), W_class, r2(b_class))
    return out
